# baseline (speedup 1.0000x reference)
import functools

import numpy as np
import jax
import jax.numpy as jnp
from jax import lax
from jax.experimental import pallas as pl
from jax.experimental.pallas import tpu as pltpu
from jax.experimental.pallas import tpu_sc as plsc

_SEQ = 200
_D = 32
_BATCH = 4096
_NW = 32
_BBLK = 512
_NBLK = _BATCH // _BBLK
_NITEM = _SEQ * _NBLK
_IPW = _NITEM // _NW
_NG = _BBLK // 128
_S_STRIDE = 4 * 32 * 8 * 128
_DT_STRIDE = 32 * 8 * 128
_FLAT = _SEQ * _S_STRIDE


def _positional_embedding():
    pos = np.arange(_SEQ, dtype=np.float32)[:, None]
    exp_sin = np.arange(0, _D, 2, dtype=np.float32) / _D * 2.0
    exp_cos = np.arange(1, _D + 1, 2, dtype=np.float32) / _D * 2.0
    sins = np.sin(pos / np.power(10000.0, exp_sin))
    coss = np.cos(pos / np.power(10000.0, exp_cos))
    pe = np.stack([sins, coss], axis=2).reshape(_SEQ, _D)
    return jnp.asarray(pe, dtype=jnp.float32)


def _body(idx_hbm, pe_hbm, table_hbm, out_hbm, idx_v, pe_v, g_v, t_v, sem):
    wid = lax.axis_index("s") * 2 + lax.axis_index("c")
    pltpu.sync_copy(pe_hbm, pe_v)


    def item_body(k, carry):
        item = wid + _NW * k
        s = item // _NBLK
        j = item % _NBLK

        pltpu.sync_copy(idx_hbm.at[pl.ds(item * _NG, _NG)], idx_v)
        for r in range(_NG):
            pltpu.make_async_copy(
                table_hbm.at[idx_v.at[r]],
                g_v.at[pl.ds(r * 128, 128)],
                sem,
            ).start()
        for r in range(_NG):
            pltpu.make_async_copy(
                table_hbm.at[idx_v.at[r]],
                g_v.at[pl.ds(r * 128, 128)],
                sem,
            ).wait()

        def row_body(b, c2):
            lanes = jax.lax.iota(jnp.int32, 16)
            dpos = (lanes >> 3) * 4096 + (lanes & 7) * 128
            col = dpos + jnp.broadcast_to((b >> 7) * 1024 + (b & 127), (16,))
            p0 = pe_v[s, pl.ds(0, 16)]
            p1 = pe_v[s, pl.ds(16, 16)]
            plsc.store_scatter(t_v, [col], g_v[b, pl.ds(0, 16)] + p0)
            plsc.store_scatter(t_v, [col + 8192], g_v[b, pl.ds(16, 16)] + p1)
            return c2

        lax.fori_loop(0, _BBLK, row_body, 0, unroll=False)

        base = s * _S_STRIDE + j * (_NG * 1024)
        for dt in range(4):
            pltpu.sync_copy(
                t_v.at[pl.ds(dt * (_NG * 1024), _NG * 1024)],
                out_hbm.at[pl.ds(base + dt * _DT_STRIDE, _NG * 1024)],
            )
        return carry

    lax.fori_loop(0, _IPW, item_body, 0, unroll=False)


@jax.jit
def _embed(idx_grouped, pe, token_table):
    mesh = plsc.VectorSubcoreMesh(core_axis_name="c", subcore_axis_name="s")
    run = functools.partial(
        pl.kernel,
        mesh=mesh,
        out_type=jax.ShapeDtypeStruct((_FLAT,), jnp.float32),
        scratch_types=[
            pltpu.VMEM((_NG, 128), jnp.int32),
            pltpu.VMEM((_SEQ, _D), jnp.float32),
            pltpu.VMEM((_BBLK, _D), jnp.float32),
            pltpu.VMEM((4 * _NG * 1024,), jnp.float32),
            pltpu.SemaphoreType.DMA,
        ],
        compiler_params=pltpu.CompilerParams(
            use_tc_tiling_on_sc=False, needs_layout_passes=False
        ),
    )(_body)
    return run(idx_grouped, pe, token_table)


def kernel(sequence, token_table):
    idx_grouped = jnp.transpose(sequence).reshape(_SEQ * _BATCH // 128, 128)
    pe = _positional_embedding()
    flat = _embed(idx_grouped, pe, token_table)
    f5 = flat.reshape(_SEQ, 4, 32, 8, 128)
    return f5.transpose(2, 4, 0, 1, 3).reshape(_BATCH, _SEQ, _D)


# speedup vs baseline: 1.1582x; 1.1582x over previous
import functools

import numpy as np
import jax
import jax.numpy as jnp
from jax import lax
from jax.experimental import pallas as pl
from jax.experimental.pallas import tpu as pltpu
from jax.experimental.pallas import tpu_sc as plsc

_SEQ = 200
_D = 32
_BATCH = 4096
_NW = 32
_BBLK = 512
_NBLK = _BATCH // _BBLK
_NITEM = _SEQ * _NBLK
_IPW = _NITEM // _NW
_NG = _BBLK // 128
_FLAT = _SEQ * _BATCH * _D


def _positional_embedding():
    pos = np.arange(_SEQ, dtype=np.float32)[:, None]
    exp_sin = np.arange(0, _D, 2, dtype=np.float32) / _D * 2.0
    exp_cos = np.arange(1, _D + 1, 2, dtype=np.float32) / _D * 2.0
    sins = np.sin(pos / np.power(10000.0, exp_sin))
    coss = np.cos(pos / np.power(10000.0, exp_cos))
    pe = np.stack([sins, coss], axis=2).reshape(_SEQ, _D)
    return jnp.asarray(pe, dtype=jnp.float32)


def _body(idx_hbm, pe_hbm, table_hbm, out_hbm, idx_v, pe_v, g_v, sem):
    wid = lax.axis_index("s") * 2 + lax.axis_index("c")
    pltpu.sync_copy(pe_hbm, pe_v)

    def item_body(k, carry):
        item = wid + _NW * k
        s = item // _NBLK
        j = item % _NBLK

        pltpu.sync_copy(idx_hbm.at[pl.ds(item * _NG, _NG)], idx_v)
        for r in range(_NG):
            pltpu.make_async_copy(
                table_hbm.at[idx_v.at[r]],
                g_v.at[pl.ds(r * 128, 128)],
                sem,
            ).start()
        for r in range(_NG):
            pltpu.make_async_copy(
                table_hbm.at[idx_v.at[r]],
                g_v.at[pl.ds(r * 128, 128)],
                sem,
            ).wait()

        def row_body(i, c2):
            p0 = pe_v[s, pl.ds(0, 16)]
            p1 = pe_v[s, pl.ds(16, 16)]
            for u in range(4):
                b = i * 4 + u
                plsc.addupdate(g_v.at[b, pl.ds(0, 16)], p0)
                plsc.addupdate(g_v.at[b, pl.ds(16, 16)], p1)
            return c2

        lax.fori_loop(0, _BBLK // 4, row_body, 0, unroll=False)

        pltpu.sync_copy(g_v, out_hbm.at[pl.ds(item * _BBLK, _BBLK)])
        return carry

    lax.fori_loop(0, _IPW, item_body, 0, unroll=False)


@jax.jit
def _embed(idx_grouped, pe, token_table):
    mesh = plsc.VectorSubcoreMesh(core_axis_name="c", subcore_axis_name="s")
    run = functools.partial(
        pl.kernel,
        mesh=mesh,
        out_type=jax.ShapeDtypeStruct((_SEQ * _BATCH, _D), jnp.float32),
        scratch_types=[
            pltpu.VMEM((_NG, 128), jnp.int32),
            pltpu.VMEM((_SEQ, _D), jnp.float32),
            pltpu.VMEM((_BBLK, _D), jnp.float32),
            pltpu.SemaphoreType.DMA,
        ],
        compiler_params=pltpu.CompilerParams(
            use_tc_tiling_on_sc=False, needs_layout_passes=False
        ),
    )(_body)
    return run(idx_grouped, pe, token_table)


def kernel(sequence, token_table):
    idx_grouped = jnp.transpose(sequence).reshape(_SEQ * _BATCH // 128, 128)
    pe = _positional_embedding()
    flat = _embed(idx_grouped, pe, token_table)
    return jnp.transpose(flat.reshape(_SEQ, _BATCH, _D), (1, 0, 2))
